# baseline (speedup 1.0000x reference)
import functools

import jax
import jax.numpy as jnp
from jax import lax
from jax.experimental import pallas as pl
from jax.experimental.pallas import tpu as pltpu
from jax.experimental.pallas import tpu_sc as plsc

B = 16384
D = 1024
NC = 2
NS = 16
NW = NC * NS
RPW = B // NW
CH = 64
NCH = RPW // CH
NBUF = 2


@functools.partial(
    pl.kernel,
    out_type=jax.ShapeDtypeStruct((B, D), jnp.float32),
    mesh=plsc.VectorSubcoreMesh(core_axis_name="c", subcore_axis_name="s"),
    scratch_types=(
        [pltpu.VMEM_SHARED((NS, CH, D), jnp.float32),
         pltpu.VMEM((CH, D), jnp.float32)]
        + [pltpu.SemaphoreType.DMA for _ in range(2 * NBUF)]
    ),
)
def _pos_copy(table_hbm, out_hbm, shared, tbuf,
              g0, g1, o0, o1):
    sid = lax.axis_index("s")
    wid = sid * NC + lax.axis_index("c")
    base = wid * RPW

    bufs = (shared.at[sid], tbuf)
    gsems = (g0, g1)
    osems = (o0, o1)
    gathers = [None] * NBUF
    out_pending = [None] * NBUF

    def start_gather(c):
        b = c % NBUF
        gathers[b] = pltpu.async_copy(
            table_hbm.at[pl.ds(base + c * CH, CH)], bufs[b], gsems[b])

    for c in range(NBUF - 1):
        start_gather(c)
    for c in range(NCH):
        b = c % NBUF
        gathers[b].wait()
        out_pending[b] = pltpu.async_copy(
            bufs[b], out_hbm.at[pl.ds(base + c * CH, CH)], osems[b])
        n = c + NBUF - 1
        if n < NCH:
            bn = n % NBUF
            if out_pending[bn] is not None:
                out_pending[bn].wait()
                out_pending[bn] = None
            start_gather(n)
    for b in range(NBUF):
        if out_pending[b] is not None:
            out_pending[b].wait()


def kernel(grid_size, pos_embed_table):
    del grid_size
    table = pos_embed_table.reshape(B, D)
    out = _pos_copy(table)
    return out.reshape(1, B, D)
